# Initial kernel scaffold; baseline (speedup 1.0000x reference)
import jax
import jax.numpy as jnp
from jax.experimental import pallas as pl


def kernel(x, table):
    raise NotImplementedError("write your pallas kernel here")


# speedup vs baseline: 1.0238x; 1.0238x over previous
import functools

import jax
import jax.numpy as jnp
from jax import lax
from jax.experimental import pallas as pl
from jax.experimental.pallas import tpu as pltpu
from jax.experimental.pallas import tpu_sc as plsc

NC = 2
NS = 16
NW = NC * NS

VOCAB = 1000000
EMBED_DIM = 32
BATCH = 16384
HIST = 50

TOTAL = BATCH * HIST
ROWS_PER_W = TOTAL // NW
GBATCH = 128
NG = ROWS_PER_W // GBATCH


def _body(x_hbm, table_hbm, out_hbm, idx_v, rows_v, gsem):
    wid = lax.axis_index("s") * NC + lax.axis_index("c")
    idx_row0 = wid * NG
    out_row0 = wid * ROWS_PER_W

    pltpu.sync_copy(x_hbm.at[pl.ds(idx_row0, NG)], idx_v)

    def step(j, carry):
        pltpu.async_copy(table_hbm.at[idx_v.at[j]], rows_v, gsem).wait()
        pltpu.sync_copy(rows_v, out_hbm.at[pl.ds(out_row0 + j * GBATCH, GBATCH)])
        return carry

    lax.fori_loop(0, NG, step, 0)


@functools.partial(
    pl.kernel,
    out_type=jax.ShapeDtypeStruct((TOTAL, EMBED_DIM), jnp.float32),
    mesh=plsc.VectorSubcoreMesh(
        core_axis_name="c", subcore_axis_name="s", num_cores=NC, num_subcores=NS
    ),
    scratch_types=[
        pltpu.VMEM((NG, GBATCH), jnp.int32),
        pltpu.VMEM((GBATCH, EMBED_DIM), jnp.float32),
        pltpu.SemaphoreType.DMA,
    ],
    compiler_params=pltpu.CompilerParams(use_tc_tiling_on_sc=False),
)
def _gather_kernel(x_hbm, table_hbm, out_hbm, idx_v, rows_v, gsem):
    _body(x_hbm, table_hbm, out_hbm, idx_v, rows_v, gsem)


def kernel(x, table):
    x2d = x.reshape(TOTAL // GBATCH, GBATCH).astype(jnp.int32)
    out = _gather_kernel(x2d, table)
    return out.reshape(BATCH, HIST, EMBED_DIM)


# speedup vs baseline: 1.1001x; 1.0746x over previous
import functools

import jax
import jax.numpy as jnp
from jax import lax
from jax.experimental import pallas as pl
from jax.experimental.pallas import tpu as pltpu
from jax.experimental.pallas import tpu_sc as plsc

NC = 2
NS = 16
NW = NC * NS

VOCAB = 1000000
EMBED_DIM = 32
BATCH = 16384
HIST = 50

TOTAL = BATCH * HIST
ROWS_PER_W = TOTAL // NW
GBATCH = 128
NG = ROWS_PER_W // GBATCH
K = 4
NGROUPS = NG // K


def _body(x_hbm, table_hbm, out_hbm, idx_v, rows_v, gsem0, gsem1, wsem0, wsem1):
    wid = lax.axis_index("s") * NC + lax.axis_index("c")
    idx_row0 = wid * NG
    out_row0 = wid * ROWS_PER_W

    pltpu.sync_copy(x_hbm.at[pl.ds(idx_row0, NG)], idx_v)

    def start_gathers(g, p, gsem):
        for b in range(K):
            pltpu.async_copy(
                table_hbm.at[idx_v.at[g * K + b]], rows_v.at[p, b], gsem
            )

    def wait_gathers(g, p, gsem):
        for b in range(K):
            pltpu.make_async_copy(
                table_hbm.at[idx_v.at[g * K + b]], rows_v.at[p, b], gsem
            ).wait()

    def start_writes(g, p, wsem):
        for b in range(K):
            pltpu.async_copy(
                rows_v.at[p, b],
                out_hbm.at[pl.ds(out_row0 + (g * K + b) * GBATCH, GBATCH)],
                wsem,
            )

    def wait_writes(g, p, wsem):
        for b in range(K):
            pltpu.make_async_copy(
                rows_v.at[p, b],
                out_hbm.at[pl.ds(out_row0 + (g * K + b) * GBATCH, GBATCH)],
                wsem,
            ).wait()

    start_gathers(0, 0, gsem0)
    start_gathers(1, 1, gsem1)
    wait_gathers(0, 0, gsem0)
    start_writes(0, 0, wsem0)

    def pair(i, carry):
        g = 1 + i * 2
        wait_gathers(g, 1, gsem1)
        start_writes(g, 1, wsem1)
        wait_writes(g - 1, 0, wsem0)
        start_gathers(g + 1, 0, gsem0)
        wait_gathers(g + 1, 0, gsem0)
        start_writes(g + 1, 0, wsem0)
        wait_writes(g, 1, wsem1)
        start_gathers(g + 2, 1, gsem1)
        return carry

    lax.fori_loop(0, (NGROUPS - 2) // 2, pair, 0)

    g_last = NGROUPS - 1
    wait_gathers(g_last, 1, gsem1)
    start_writes(g_last, 1, wsem1)
    wait_writes(g_last - 1, 0, wsem0)
    wait_writes(g_last, 1, wsem1)


@functools.partial(
    pl.kernel,
    out_type=jax.ShapeDtypeStruct((TOTAL, EMBED_DIM), jnp.float32),
    mesh=plsc.VectorSubcoreMesh(
        core_axis_name="c", subcore_axis_name="s", num_cores=NC, num_subcores=NS
    ),
    scratch_types=[
        pltpu.VMEM((NG, GBATCH), jnp.int32),
        pltpu.VMEM((2, K, GBATCH, EMBED_DIM), jnp.float32),
        pltpu.SemaphoreType.DMA,
        pltpu.SemaphoreType.DMA,
        pltpu.SemaphoreType.DMA,
        pltpu.SemaphoreType.DMA,
    ],
    compiler_params=pltpu.CompilerParams(use_tc_tiling_on_sc=False),
)
def _gather_kernel(x_hbm, table_hbm, out_hbm, idx_v, rows_v, g0, g1, w0, w1):
    _body(x_hbm, table_hbm, out_hbm, idx_v, rows_v, g0, g1, w0, w1)


def kernel(x, table):
    x2d = x.reshape(TOTAL // GBATCH, GBATCH).astype(jnp.int32)
    out = _gather_kernel(x2d, table)
    return out.reshape(BATCH, HIST, EMBED_DIM)


# speedup vs baseline: 1.5363x; 1.3965x over previous
import functools

import jax
import jax.numpy as jnp
from jax import lax
from jax.experimental import pallas as pl
from jax.experimental.pallas import tpu as pltpu
from jax.experimental.pallas import tpu_sc as plsc

NC = 2
NS = 16
NW = NC * NS
LANES = 16

VOCAB = 1000000
EMBED_DIM = 32
BATCH = 16384
HIST = 50

TOTAL = BATCH * HIST
GBATCH = 128
NBLK = TOTAL // GBATCH
BLK_PER_W = NBLK // NW
BBLK = BATCH // GBATCH
DR = EMBED_DIM // 8


def _transpose_block(rows, tbuf, row_idx):
    for r in range(DR):
        for s in range(8):
            d = r * 8 + s
            col_idx = jnp.full((LANES,), d, jnp.int32)
            for lg in range(8):
                vals = plsc.load_gather(rows, [row_idx[lg], col_idx])
                tbuf[r, s, pl.ds(lg * LANES, LANES)] = vals


def _body(x_hbm, table_hbm, out_hbm, idx_v, rows_v, tbuf_v, gsem, wsem):
    wid = lax.axis_index("s") * NC + lax.axis_index("c")
    blk0 = wid * BLK_PER_W

    pltpu.sync_copy(x_hbm.at[pl.ds(blk0, BLK_PER_W)], idx_v)

    iota = lax.iota(jnp.int32, LANES)
    row_idx = [iota + lg * LANES for lg in range(8)]

    def start_gather(j, p):
        pltpu.async_copy(table_hbm.at[idx_v.at[j]], rows_v.at[p], gsem)

    def wait_gather(j, p):
        pltpu.make_async_copy(
            table_hbm.at[idx_v.at[j]], rows_v.at[p], gsem
        ).wait()

    def start_writes(j, p):
        blk = blk0 + j
        h = blk // BBLK
        cb = blk - h * BBLK
        for r in range(DR):
            pltpu.async_copy(tbuf_v.at[p, r], out_hbm.at[h, r, cb], wsem)

    def wait_writes(j, p):
        blk = blk0 + j
        h = blk // BBLK
        cb = blk - h * BBLK
        for r in range(DR):
            pltpu.make_async_copy(
                tbuf_v.at[p, r], out_hbm.at[h, r, cb], wsem
            ).wait()

    def transpose(p):
        _transpose_block(rows_v.at[p], tbuf_v.at[p], row_idx)

    start_gather(0, 0)
    wait_gather(0, 0)
    start_gather(1, 1)
    transpose(0)
    start_writes(0, 0)
    wait_gather(1, 1)
    start_gather(2, 0)
    transpose(1)
    start_writes(1, 1)

    def pair(i, carry):
        j = 2 + i * 2
        wait_gather(j, 0)
        wait_writes(j - 2, 0)
        start_gather(j + 1, 1)
        transpose(0)
        start_writes(j, 0)
        wait_gather(j + 1, 1)
        wait_writes(j - 1, 1)
        start_gather(j + 2, 0)
        transpose(1)
        start_writes(j + 1, 1)
        return carry

    lax.fori_loop(0, (BLK_PER_W - 4) // 2, pair, 0)

    j = BLK_PER_W - 2
    start_gather(j + 1, 1)
    wait_gather(j, 0)
    wait_writes(j - 2, 0)
    transpose(0)
    start_writes(j, 0)
    wait_gather(j + 1, 1)
    wait_writes(j - 1, 1)
    transpose(1)
    start_writes(j + 1, 1)
    wait_writes(j, 0)
    wait_writes(j + 1, 1)


@functools.partial(
    pl.kernel,
    out_type=jax.ShapeDtypeStruct((HIST, DR, BBLK, 8, GBATCH), jnp.float32),
    mesh=plsc.VectorSubcoreMesh(
        core_axis_name="c", subcore_axis_name="s", num_cores=NC, num_subcores=NS
    ),
    scratch_types=[
        pltpu.VMEM((BLK_PER_W, GBATCH), jnp.int32),
        pltpu.VMEM((2, GBATCH, EMBED_DIM), jnp.float32),
        pltpu.VMEM((2, DR, 8, GBATCH), jnp.float32),
        pltpu.SemaphoreType.DMA,
        pltpu.SemaphoreType.DMA,
    ],
    compiler_params=pltpu.CompilerParams(
        use_tc_tiling_on_sc=False, needs_layout_passes=False
    ),
)
def _gather_kernel(x_hbm, table_hbm, out_hbm, idx_v, rows_v, tbuf_v, g, w):
    _body(x_hbm, table_hbm, out_hbm, idx_v, rows_v, tbuf_v, g, w)


def kernel(x, table):
    xt = jnp.transpose(x).astype(jnp.int32).reshape(NBLK, GBATCH)
    res = _gather_kernel(xt, table)
    return res.transpose(2, 4, 0, 1, 3).reshape(BATCH, HIST, EMBED_DIM)
